# baseline (speedup 1.0000x reference)
import functools

import jax
import jax.numpy as jnp
from jax import lax
from jax.experimental import pallas as pl
from jax.experimental.pallas import tpu as pltpu
from jax.experimental.pallas import tpu_sc as plsc

EMBED = 128
CHUNK = 128
BLOCK = 2048
TIME_OFF = 16
TD_OFF = 64


def _sc_gather_body(n_tokens, loc_ids, loc_t, out, loc_i, loc_rows,
                    sem_i, sem_g, sem_o):
    info = plsc.get_sparse_core_info()
    nw = info.num_cores * info.num_subcores
    wid = lax.axis_index("s") * info.num_cores + lax.axis_index("c")
    per_tile = n_tokens // nw
    base = wid * per_tile
    nc = per_tile // CHUNK

    def ids_start(c, s):
        return pltpu.async_copy(
            loc_ids.at[pl.ds(base + c * CHUNK, CHUNK)], loc_i.at[s], sem_i.at[s])

    def gather_start(s):
        return pltpu.async_copy(loc_t.at[loc_i.at[s]], loc_rows.at[s],
                                sem_g.at[s])

    def out_start(c, s):
        return pltpu.async_copy(loc_rows.at[s],
                                out.at[pl.ds(base + c * CHUNK, CHUNK)],
                                sem_o.at[s])

    ids_start(0, 0).wait()
    gather_start(0)
    ids_start(1, 1)

    @pl.loop(0, nc - 2, step=2)
    def pipe(c):
        for k in range(2):
            s = k
            t = 1 - k
            pltpu.make_async_copy(loc_t.at[loc_i.at[s]], loc_rows.at[s],
                                  sem_g.at[s]).wait()
            out_start(c + k, s)
            pltpu.make_async_copy(loc_ids.at[pl.ds(0, CHUNK)], loc_i.at[t],
                                  sem_i.at[t]).wait()
            gather_start(t)
            ids_start(c + k + 2, s)
            pltpu.make_async_copy(loc_rows.at[s],
                                  out.at[pl.ds(0, CHUNK)], sem_o.at[s]).wait()

    pltpu.make_async_copy(loc_t.at[loc_i.at[0]], loc_rows.at[0],
                          sem_g.at[0]).wait()
    out_start(nc - 2, 0)
    pltpu.make_async_copy(loc_ids.at[pl.ds(0, CHUNK)], loc_i.at[1],
                          sem_i.at[1]).wait()
    gather_start(1)
    pltpu.make_async_copy(loc_t.at[loc_i.at[1]], loc_rows.at[1],
                          sem_g.at[1]).wait()
    out_start(nc - 1, 1)
    pltpu.make_async_copy(loc_rows.at[0], out.at[pl.ds(0, CHUNK)],
                          sem_o.at[0]).wait()
    pltpu.make_async_copy(loc_rows.at[1], out.at[pl.ds(0, CHUNK)],
                          sem_o.at[1]).wait()


def _tc_body(loc_ref, day_ref, time_ref, td_ref, ctab_ref, scale_ref,
             bias_ref, out_ref):
    ci = lax.broadcasted_iota(jnp.int32, (BLOCK, EMBED), 1)
    onehot = (ci == day_ref[...]) | (ci == time_ref[...]) | (ci == td_ref[...])
    small = jnp.dot(onehot.astype(jnp.float32), ctab_ref[...],
                    preferred_element_type=jnp.float32)
    x = loc_ref[...] + small
    mean = jnp.mean(x, axis=-1, keepdims=True)
    xc = x - mean
    var = jnp.mean(xc * xc, axis=-1, keepdims=True)
    inv = lax.rsqrt(var + jnp.float32(1e-6))
    out_ref[...] = xc * inv * scale_ref[...] + bias_ref[...]


def kernel(day_ids, time_ids, location_ids, timedelta_ids,
           day_table, time_table, location_table, timedelta_table,
           ln_scale, ln_bias):
    b, l = day_ids.shape
    n = b * l
    flat = lambda x: x.reshape(n).astype(jnp.int32)

    mesh = plsc.VectorSubcoreMesh(core_axis_name="c", subcore_axis_name="s")
    gather = pl.kernel(
        functools.partial(_sc_gather_body, n),
        out_type=jax.ShapeDtypeStruct((n, EMBED), jnp.float32),
        mesh=mesh,
        scratch_types=[
            pltpu.VMEM((2, CHUNK), jnp.int32),
            pltpu.VMEM((2, CHUNK, EMBED), jnp.float32),
            pltpu.SemaphoreType.DMA((2,)),
            pltpu.SemaphoreType.DMA((2,)),
            pltpu.SemaphoreType.DMA((2,)),
        ],
        compiler_params=pltpu.CompilerParams(needs_layout_passes=False),
    )
    loc_emb = gather(flat(location_ids), location_table)

    ctab = jnp.zeros((EMBED, EMBED), jnp.float32)
    ctab = ctab.at[0:day_table.shape[0]].set(day_table)
    ctab = ctab.at[TIME_OFF:TIME_OFF + time_table.shape[0]].set(time_table)
    ctab = ctab.at[TD_OFF:TD_OFF + timedelta_table.shape[0]].set(
        timedelta_table)

    ids2 = lambda x, o: (x.reshape(n, 1).astype(jnp.int32) + o)
    tok_spec = pl.BlockSpec((BLOCK, EMBED), lambda i: (i, 0))
    id_spec = pl.BlockSpec((BLOCK, 1), lambda i: (i, 0))
    full2 = lambda r: pl.BlockSpec((r, EMBED), lambda i: (0, 0))

    out = pl.pallas_call(
        _tc_body,
        grid=(n // BLOCK,),
        in_specs=[tok_spec, id_spec, id_spec, id_spec, full2(EMBED),
                  full2(1), full2(1)],
        out_specs=tok_spec,
        out_shape=jax.ShapeDtypeStruct((n, EMBED), jnp.float32),
        compiler_params=pltpu.CompilerParams(
            dimension_semantics=("arbitrary",),
        ),
    )(loc_emb, ids2(day_ids, 0), ids2(time_ids, TIME_OFF),
      ids2(timedelta_ids, TD_OFF), ctab,
      ln_scale.reshape(1, EMBED), ln_bias.reshape(1, EMBED))
    return out.reshape(b, l, EMBED)
